# baseline (speedup 1.0000x reference)
import functools

import jax
import jax.numpy as jnp
from jax import lax
from jax.experimental import pallas as pl
from jax.experimental.pallas import tpu as pltpu
from jax.experimental.pallas import tpu_sc as plsc

N = 10000
E = 160000
D = 256
G = 64
EPS = 1e-5

NC = 2
NS = 16
CHUNK = 125
GRP = 16
ROWS_E = E // CHUNK
RPT_AGG = ROWS_E // NS
RPT_DEG = ROWS_E // (NC * NS)
NPT = 624
TAIL0 = NS * NPT
TAILN = N - TAIL0
DEGW = 128
HALF = D // 2
BR = 1000
NBLK = N // BR

_sc_mesh = plsc.VectorSubcoreMesh(
    core_axis_name="c", subcore_axis_name="s", num_cores=NC, num_subcores=NS)


def _part_copy(src, dst, s):
    r0 = s * NPT
    pltpu.sync_copy(src.at[pl.ds(r0, NPT)], dst.at[pl.ds(r0, NPT)])

    @pl.when(s == NS - 1)
    def _():
        pltpu.sync_copy(src.at[pl.ds(TAIL0, TAILN)], dst.at[pl.ds(TAIL0, TAILN)])


@functools.partial(
    pl.kernel,
    out_type=(jax.ShapeDtypeStruct((N, DEGW), jnp.float32),
              jax.ShapeDtypeStruct((N, DEGW), jnp.float32)),
    mesh=_sc_mesh,
    scratch_types=[
        pltpu.VMEM((RPT_DEG, CHUNK), jnp.int32),
        pltpu.VMEM((CHUNK, DEGW), jnp.float32),
        pltpu.VMEM_SHARED((N, DEGW), jnp.float32),
    ],
)
def _deg_kernel(dst_hbm, zeros_hbm, ones_hbm, out0, out1, idx_v, ones_v, acc):
    c = lax.axis_index("c")
    s = lax.axis_index("s")
    w = s * NC + c
    _part_copy(zeros_hbm, acc, s)
    pltpu.sync_copy(ones_hbm, ones_v)
    pltpu.sync_copy(dst_hbm.at[pl.ds(w * RPT_DEG, RPT_DEG)], idx_v)
    plsc.subcore_barrier()

    def body(j, carry):
        pltpu.sync_copy(ones_v, acc.at[idx_v.at[j]], add=True)
        return carry

    lax.fori_loop(0, RPT_DEG, body, 0)
    plsc.subcore_barrier()

    @pl.when(c == 0)
    def _():
        _part_copy(acc, out0, s)

    @pl.when(c == 1)
    def _():
        _part_copy(acc, out1, s)


@functools.partial(
    pl.kernel,
    out_type=(jax.ShapeDtypeStruct((N, HALF), jnp.float32),
              jax.ShapeDtypeStruct((N, HALF), jnp.float32)),
    mesh=_sc_mesh,
    scratch_types=[
        pltpu.VMEM((GRP, CHUNK), jnp.int32),
        pltpu.VMEM((GRP, CHUNK), jnp.int32),
        pltpu.VMEM((2, CHUNK, HALF), jnp.float32),
        pltpu.VMEM_SHARED((N, HALF), jnp.float32),
        pltpu.SemaphoreType.DMA,
        pltpu.SemaphoreType.DMA,
        pltpu.SemaphoreType.DMA,
        pltpu.SemaphoreType.DMA,
    ],
)
def _agg_kernel(hs0, hs1, src_hbm, dst_hbm, out0, out1,
                src_v, dst_v, rows_v, acc, sem0, sem1, ssem0, ssem1):
    c = lax.axis_index("c")
    s = lax.axis_index("s")
    e0 = s * RPT_AGG

    @pl.when(c == 0)
    def _():
        _part_copy(hs0, acc, s)

    @pl.when(c == 1)
    def _():
        _part_copy(hs1, acc, s)

    plsc.subcore_barrier()

    def _pipe(hs_ref):
        def group(g, carry):
            pltpu.sync_copy(src_hbm.at[pl.ds(e0 + g * GRP, GRP)], src_v)
            pltpu.sync_copy(dst_hbm.at[pl.ds(e0 + g * GRP, GRP)], dst_v)
            pltpu.async_copy(hs_ref.at[src_v.at[0]], rows_v.at[0], sem0)
            pltpu.async_copy(hs_ref.at[src_v.at[1]], rows_v.at[1], sem1)

            def pair(p, carry2):
                j = 2 * p
                pltpu.make_async_copy(
                    hs_ref.at[src_v.at[j]], rows_v.at[0], sem0).wait()
                pltpu.async_copy(rows_v.at[0], acc.at[dst_v.at[j]], ssem0,
                                 add=True)
                pltpu.make_async_copy(
                    hs_ref.at[src_v.at[j + 1]], rows_v.at[1], sem1).wait()
                pltpu.async_copy(rows_v.at[1], acc.at[dst_v.at[j + 1]], ssem1,
                                 add=True)

                pltpu.make_async_copy(
                    rows_v.at[0], acc.at[dst_v.at[j]], ssem0).wait()

                @pl.when(p < GRP // 2 - 1)
                def _():
                    pltpu.async_copy(
                        hs_ref.at[src_v.at[j + 2]], rows_v.at[0], sem0)

                pltpu.make_async_copy(
                    rows_v.at[1], acc.at[dst_v.at[j + 1]], ssem1).wait()

                @pl.when(p < GRP // 2 - 1)
                def _():
                    pltpu.async_copy(
                        hs_ref.at[src_v.at[j + 3]], rows_v.at[1], sem1)

                return carry2

            lax.fori_loop(0, GRP // 2, pair, 0)
            return carry

        lax.fori_loop(0, RPT_AGG // GRP, group, 0)

    @pl.when(c == 0)
    def _():
        _pipe(hs0)

    @pl.when(c == 1)
    def _():
        _pipe(hs1)

    plsc.subcore_barrier()

    @pl.when(c == 0)
    def _():
        _part_copy(acc, out0, s)

    @pl.when(c == 1)
    def _():
        _part_copy(acc, out1, s)


def _dinv_of(dA_ref, dB_ref):
    deg = dA_ref[:, :1] + dB_ref[:, :1] + 1.0
    return lax.rsqrt(deg)


def _mm1_body(x_ref, w_ref, dA_ref, dB_ref, o0_ref, o1_ref):
    dinv = _dinv_of(dA_ref, dB_ref)
    h = jnp.dot(x_ref[...], w_ref[...], preferred_element_type=jnp.float32)
    hs = h * dinv
    o0_ref[...] = hs[:, :HALF]
    o1_ref[...] = hs[:, HALF:]


def _mm1(x, W, dA, dB):
    return pl.pallas_call(
        _mm1_body,
        grid=(NBLK,),
        in_specs=[
            pl.BlockSpec((BR, D), lambda i: (i, 0)),
            pl.BlockSpec((D, D), lambda i: (0, 0)),
            pl.BlockSpec((BR, DEGW), lambda i: (i, 0)),
            pl.BlockSpec((BR, DEGW), lambda i: (i, 0)),
        ],
        out_specs=[pl.BlockSpec((BR, HALF), lambda i: (i, 0))] * 2,
        out_shape=[jax.ShapeDtypeStruct((N, HALF), jnp.float32)] * 2,
    )(x, W, dA, dB)


def _z_of(a0_ref, a1_ref, dinv, b_ref):
    z = jnp.concatenate([a0_ref[...], a1_ref[...]], axis=1)
    return z * dinv + b_ref[...]


def _stats_body(a0_ref, a1_ref, dA_ref, dB_ref, b_ref, o_ref):
    i = pl.program_id(0)
    z = _z_of(a0_ref, a1_ref, _dinv_of(dA_ref, dB_ref), b_ref)
    ps = jnp.sum(z, axis=0, keepdims=True)
    pss = jnp.sum(z * z, axis=0, keepdims=True)
    part = jnp.concatenate([ps, pss, jnp.zeros((6, D), jnp.float32)], axis=0)

    @pl.when(i == 0)
    def _():
        o_ref[...] = part

    @pl.when(i != 0)
    def _():
        o_ref[...] += part


def _stats(a0, a1, dA, dB, b):
    return pl.pallas_call(
        _stats_body,
        grid=(NBLK,),
        in_specs=[
            pl.BlockSpec((BR, HALF), lambda i: (i, 0)),
            pl.BlockSpec((BR, HALF), lambda i: (i, 0)),
            pl.BlockSpec((BR, DEGW), lambda i: (i, 0)),
            pl.BlockSpec((BR, DEGW), lambda i: (i, 0)),
            pl.BlockSpec((1, D), lambda i: (0, 0)),
        ],
        out_specs=pl.BlockSpec((8, D), lambda i: (0, 0)),
        out_shape=jax.ShapeDtypeStruct((8, D), jnp.float32),
    )(a0, a1, dA, dB, b)


def _bn_relu(z, g_ref, be_ref, st_ref):
    m = st_ref[0:1, :] * (1.0 / N)
    v = st_ref[1:2, :] * (1.0 / N) - m * m
    return jnp.maximum(g_ref[...] * (z - m) * lax.rsqrt(v + EPS) + be_ref[...], 0.0)


def _mmn_body(a0_ref, a1_ref, dA_ref, dB_ref, b_ref, g_ref, be_ref, st_ref,
              w_ref, o0_ref, o1_ref):
    dinv = _dinv_of(dA_ref, dB_ref)
    y = _bn_relu(_z_of(a0_ref, a1_ref, dinv, b_ref), g_ref, be_ref, st_ref)
    h = jnp.dot(y, w_ref[...], preferred_element_type=jnp.float32)
    hs = h * dinv
    o0_ref[...] = hs[:, :HALF]
    o1_ref[...] = hs[:, HALF:]


def _mmn(a0, a1, dA, dB, b, g, be, st, W):
    return pl.pallas_call(
        _mmn_body,
        grid=(NBLK,),
        in_specs=[
            pl.BlockSpec((BR, HALF), lambda i: (i, 0)),
            pl.BlockSpec((BR, HALF), lambda i: (i, 0)),
            pl.BlockSpec((BR, DEGW), lambda i: (i, 0)),
            pl.BlockSpec((BR, DEGW), lambda i: (i, 0)),
            pl.BlockSpec((1, D), lambda i: (0, 0)),
            pl.BlockSpec((1, D), lambda i: (0, 0)),
            pl.BlockSpec((1, D), lambda i: (0, 0)),
            pl.BlockSpec((8, D), lambda i: (0, 0)),
            pl.BlockSpec((D, D), lambda i: (0, 0)),
        ],
        out_specs=[pl.BlockSpec((BR, HALF), lambda i: (i, 0))] * 2,
        out_shape=[jax.ShapeDtypeStruct((N, HALF), jnp.float32)] * 2,
    )(a0, a1, dA, dB, b, g, be, st, W)


def _head_body(a0_ref, a1_ref, dA_ref, dB_ref, b_ref, g_ref, be_ref, st_ref,
               bt_ref, wf1_ref, bf1_ref, wf2_ref, bf2_ref, o_ref,
               pool_acc, cnt_acc):
    i = pl.program_id(0)
    dinv = _dinv_of(dA_ref, dB_ref)
    y = _bn_relu(_z_of(a0_ref, a1_ref, dinv, b_ref), g_ref, be_ref, st_ref)
    bt = jnp.broadcast_to(bt_ref[...][0], (G, BR))
    oh = (bt == lax.broadcasted_iota(jnp.int32, (G, BR), 0)).astype(jnp.float32)
    pool_part = jnp.dot(oh, y, preferred_element_type=jnp.float32)
    cnt_part = jnp.broadcast_to(jnp.sum(oh, axis=1, keepdims=True), (G, DEGW))

    @pl.when(i == 0)
    def _():
        pool_acc[...] = pool_part
        cnt_acc[...] = cnt_part

    @pl.when(i != 0)
    def _():
        pool_acc[...] += pool_part
        cnt_acc[...] += cnt_part

    @pl.when(i == NBLK - 1)
    def _():
        p = pool_acc[...] / jnp.maximum(cnt_acc[:, :1], 1.0)
        q = jnp.maximum(
            jnp.dot(p, wf1_ref[...], preferred_element_type=jnp.float32)
            + bf1_ref[...], 0.0)
        o_ref[...] = (jnp.dot(q, wf2_ref[...], preferred_element_type=jnp.float32)
                      + bf2_ref[...])


def _head(a0, a1, dA, dB, b, g, be, st, bt3, Wf1, bf1, Wf2, bf2):
    return pl.pallas_call(
        _head_body,
        grid=(NBLK,),
        in_specs=[
            pl.BlockSpec((BR, HALF), lambda i: (i, 0)),
            pl.BlockSpec((BR, HALF), lambda i: (i, 0)),
            pl.BlockSpec((BR, DEGW), lambda i: (i, 0)),
            pl.BlockSpec((BR, DEGW), lambda i: (i, 0)),
            pl.BlockSpec((1, D), lambda i: (0, 0)),
            pl.BlockSpec((1, D), lambda i: (0, 0)),
            pl.BlockSpec((1, D), lambda i: (0, 0)),
            pl.BlockSpec((8, D), lambda i: (0, 0)),
            pl.BlockSpec((1, 1, BR), lambda i: (i, 0, 0)),
            pl.BlockSpec((D, D), lambda i: (0, 0)),
            pl.BlockSpec((1, D), lambda i: (0, 0)),
            pl.BlockSpec((D, 1), lambda i: (0, 0)),
            pl.BlockSpec((1, 1), lambda i: (0, 0)),
        ],
        out_specs=pl.BlockSpec((G, 1), lambda i: (0, 0)),
        out_shape=jax.ShapeDtypeStruct((G, 1), jnp.float32),
        scratch_shapes=[
            pltpu.VMEM((G, D), jnp.float32),
            pltpu.VMEM((G, DEGW), jnp.float32),
        ],
    )(a0, a1, dA, dB, b, g, be, st, bt3, Wf1, bf1, Wf2, bf2)


def kernel(x, edge_index, batch, W1, b1, g1, be1, W2, b2, g2, be2,
           W3, b3, g3, be3, Wf1, bf1, Wf2, bf2):
    src2d = edge_index[0].reshape(ROWS_E, CHUNK)
    dst2d = edge_index[1].reshape(ROWS_E, CHUNK)
    zeros_nd = jnp.zeros((N, DEGW), jnp.float32)
    ones_cd = jnp.ones((CHUNK, DEGW), jnp.float32)
    bt3 = batch.reshape(NBLK, 1, BR)
    b1r, g1r, be1r = b1.reshape(1, D), g1.reshape(1, D), be1.reshape(1, D)
    b2r, g2r, be2r = b2.reshape(1, D), g2.reshape(1, D), be2.reshape(1, D)
    b3r, g3r, be3r = b3.reshape(1, D), g3.reshape(1, D), be3.reshape(1, D)

    dA, dB = _deg_kernel(dst2d, zeros_nd, ones_cd)

    hs0, hs1 = _mm1(x, W1, dA, dB)
    a0, a1 = _agg_kernel(hs0, hs1, src2d, dst2d)
    st1 = _stats(a0, a1, dA, dB, b1r)

    hs0, hs1 = _mmn(a0, a1, dA, dB, b1r, g1r, be1r, st1, W2)
    a0, a1 = _agg_kernel(hs0, hs1, src2d, dst2d)
    st2 = _stats(a0, a1, dA, dB, b2r)

    hs0, hs1 = _mmn(a0, a1, dA, dB, b2r, g2r, be2r, st2, W3)
    a0, a1 = _agg_kernel(hs0, hs1, src2d, dst2d)
    st3 = _stats(a0, a1, dA, dB, b3r)

    return _head(a0, a1, dA, dB, b3r, g3r, be3r, st3, bt3,
                 Wf1, bf1.reshape(1, D), Wf2, bf2.reshape(1, 1))


# speedup vs baseline: 1.0593x; 1.0593x over previous
import functools

import jax
import jax.numpy as jnp
from jax import lax
from jax.experimental import pallas as pl
from jax.experimental.pallas import tpu as pltpu
from jax.experimental.pallas import tpu_sc as plsc

N = 10000
E = 160000
D = 256
G = 64
EPS = 1e-5

NC = 2
NS = 16
CHUNK = 125
GRP = 16
ROWS_E = E // CHUNK
RPT_AGG = ROWS_E // NS
RPT_DEG = ROWS_E // (NC * NS)
NPT = 624
TAIL0 = NS * NPT
TAILN = N - TAIL0
DEGW = 128
DEGS = 8
HALF = D // 2
BR = 1000
NBLK = N // BR

_sc_mesh = plsc.VectorSubcoreMesh(
    core_axis_name="c", subcore_axis_name="s", num_cores=NC, num_subcores=NS)


def _part_copy(src, dst, s):
    r0 = s * NPT
    pltpu.sync_copy(src.at[pl.ds(r0, NPT)], dst.at[pl.ds(r0, NPT)])

    @pl.when(s == NS - 1)
    def _():
        pltpu.sync_copy(src.at[pl.ds(TAIL0, TAILN)], dst.at[pl.ds(TAIL0, TAILN)])


@functools.partial(
    pl.kernel,
    out_type=(jax.ShapeDtypeStruct((N, DEGW), jnp.float32),
              jax.ShapeDtypeStruct((N, DEGW), jnp.float32)),
    mesh=_sc_mesh,
    scratch_types=[
        pltpu.VMEM((RPT_DEG, CHUNK), jnp.int32),
        pltpu.VMEM((CHUNK, DEGW), jnp.float32),
        pltpu.VMEM_SHARED((N, DEGW), jnp.float32),
    ],
)
def _deg_kernel(dst_hbm, zeros_hbm, ones_hbm, out0, out1, idx_v, ones_v, acc):
    c = lax.axis_index("c")
    s = lax.axis_index("s")
    w = s * NC + c
    _part_copy(zeros_hbm, acc, s)
    pltpu.sync_copy(ones_hbm, ones_v)
    pltpu.sync_copy(dst_hbm.at[pl.ds(w * RPT_DEG, RPT_DEG)], idx_v)
    plsc.subcore_barrier()

    def body(j, carry):
        pltpu.sync_copy(ones_v, acc.at[idx_v.at[j]], add=True)
        return carry

    lax.fori_loop(0, RPT_DEG, body, 0)
    plsc.subcore_barrier()

    @pl.when(c == 0)
    def _():
        _part_copy(acc, out0, s)

    @pl.when(c == 1)
    def _():
        _part_copy(acc, out1, s)


@functools.partial(
    pl.kernel,
    out_type=(jax.ShapeDtypeStruct((N, HALF), jnp.float32),
              jax.ShapeDtypeStruct((N, HALF), jnp.float32)),
    mesh=_sc_mesh,
    scratch_types=[
        pltpu.VMEM((GRP, CHUNK), jnp.int32),
        pltpu.VMEM((GRP, CHUNK), jnp.int32),
        pltpu.VMEM((2, CHUNK, HALF), jnp.float32),
        pltpu.VMEM_SHARED((N, HALF), jnp.float32),
        pltpu.SemaphoreType.DMA,
        pltpu.SemaphoreType.DMA,
    ],
)
def _agg_kernel(hs0, hs1, src_hbm, dst_hbm, out0, out1,
                src_v, dst_v, rows_v, acc, sem0, sem1):
    c = lax.axis_index("c")
    s = lax.axis_index("s")
    e0 = s * RPT_AGG

    @pl.when(c == 0)
    def _():
        _part_copy(hs0, acc, s)

    @pl.when(c == 1)
    def _():
        _part_copy(hs1, acc, s)

    plsc.subcore_barrier()

    def _pipe(hs_ref):
        def group(g, carry):
            pltpu.sync_copy(src_hbm.at[pl.ds(e0 + g * GRP, GRP)], src_v)
            pltpu.sync_copy(dst_hbm.at[pl.ds(e0 + g * GRP, GRP)], dst_v)
            pltpu.async_copy(hs_ref.at[src_v.at[0]], rows_v.at[0], sem0)

            def pair(p, carry2):
                j = 2 * p
                pltpu.make_async_copy(
                    hs_ref.at[src_v.at[j]], rows_v.at[0], sem0).wait()
                pltpu.async_copy(hs_ref.at[src_v.at[j + 1]], rows_v.at[1], sem1)
                pltpu.sync_copy(rows_v.at[0], acc.at[dst_v.at[j]], add=True)

                pltpu.make_async_copy(
                    hs_ref.at[src_v.at[j + 1]], rows_v.at[1], sem1).wait()

                @pl.when(p < GRP // 2 - 1)
                def _():
                    pltpu.async_copy(
                        hs_ref.at[src_v.at[j + 2]], rows_v.at[0], sem0)

                pltpu.sync_copy(rows_v.at[1], acc.at[dst_v.at[j + 1]], add=True)
                return carry2

            lax.fori_loop(0, GRP // 2, pair, 0)
            return carry

        lax.fori_loop(0, RPT_AGG // GRP, group, 0)

    @pl.when(c == 0)
    def _():
        _pipe(hs0)

    @pl.when(c == 1)
    def _():
        _pipe(hs1)

    plsc.subcore_barrier()

    @pl.when(c == 0)
    def _():
        _part_copy(acc, out0, s)

    @pl.when(c == 1)
    def _():
        _part_copy(acc, out1, s)


def _dot(a, b):
    return jnp.dot(a, b, preferred_element_type=jnp.float32)


def _dinv_of(dA_ref, dB_ref):
    deg = dA_ref[:, :1] + dB_ref[:, :1] + 1.0
    return lax.rsqrt(deg)


def _mm1_body(x_ref, w_ref, dA_ref, dB_ref, o0_ref, o1_ref):
    dinv = _dinv_of(dA_ref, dB_ref)
    h = _dot(x_ref[...], w_ref[...])
    hs = h * dinv
    o0_ref[...] = hs[:, :HALF]
    o1_ref[...] = hs[:, HALF:]


def _mm1(x, W, dA, dB):
    return pl.pallas_call(
        _mm1_body,
        grid=(NBLK,),
        in_specs=[
            pl.BlockSpec((BR, D), lambda i: (i, 0)),
            pl.BlockSpec((D, D), lambda i: (0, 0)),
            pl.BlockSpec((BR, DEGS), lambda i: (i, 0)),
            pl.BlockSpec((BR, DEGS), lambda i: (i, 0)),
        ],
        out_specs=[pl.BlockSpec((BR, HALF), lambda i: (i, 0))] * 2,
        out_shape=[jax.ShapeDtypeStruct((N, HALF), jnp.float32)] * 2,
    )(x, W, dA, dB)


def _z_of(a0_ref, a1_ref, dinv, b_ref):
    z = jnp.concatenate([a0_ref[...], a1_ref[...]], axis=1)
    return z * dinv + b_ref[...]


def _stats_part(z):
    ps = jnp.sum(z, axis=0, keepdims=True)
    pss = jnp.sum(z * z, axis=0, keepdims=True)
    return jnp.concatenate([ps, pss, jnp.zeros((6, D), jnp.float32)], axis=0)


def _accum_stats(i, z, st_acc):
    @pl.when(i == 0)
    def _():
        st_acc[...] = _stats_part(z)

    @pl.when((i > 0) & (i < NBLK))
    def _():
        st_acc[...] += _stats_part(z)


def _bn_relu(z, g_ref, be_ref, st):
    m = st[0:1, :] * (1.0 / N)
    v = st[1:2, :] * (1.0 / N) - m * m
    return jnp.maximum(g_ref[...] * (z - m) * lax.rsqrt(v + EPS) + be_ref[...], 0.0)


def _mmn_body(a0_ref, a1_ref, dA_ref, dB_ref, b_ref, g_ref, be_ref,
              w_ref, o0_ref, o1_ref, st_acc):
    i = pl.program_id(0)
    dinv = _dinv_of(dA_ref, dB_ref)
    z = _z_of(a0_ref, a1_ref, dinv, b_ref)
    _accum_stats(i, z, st_acc)

    @pl.when(i >= NBLK)
    def _():
        y = _bn_relu(z, g_ref, be_ref, st_acc[...])
        h = _dot(y, w_ref[...])
        hs = h * dinv
        o0_ref[...] = hs[:, :HALF]
        o1_ref[...] = hs[:, HALF:]


def _mmn(a0, a1, dA, dB, b, g, be, W):
    return pl.pallas_call(
        _mmn_body,
        grid=(2 * NBLK,),
        in_specs=[
            pl.BlockSpec((BR, HALF), lambda i: (i % NBLK, 0)),
            pl.BlockSpec((BR, HALF), lambda i: (i % NBLK, 0)),
            pl.BlockSpec((BR, DEGS), lambda i: (i % NBLK, 0)),
            pl.BlockSpec((BR, DEGS), lambda i: (i % NBLK, 0)),
            pl.BlockSpec((1, D), lambda i: (0, 0)),
            pl.BlockSpec((1, D), lambda i: (0, 0)),
            pl.BlockSpec((1, D), lambda i: (0, 0)),
            pl.BlockSpec((D, D), lambda i: (0, 0)),
        ],
        out_specs=[pl.BlockSpec((BR, HALF), lambda i: (i % NBLK, 0))] * 2,
        out_shape=[jax.ShapeDtypeStruct((N, HALF), jnp.float32)] * 2,
        scratch_shapes=[pltpu.VMEM((8, D), jnp.float32)],
    )(a0, a1, dA, dB, b, g, be, W)


def _head_body(a0_ref, a1_ref, dA_ref, dB_ref, b_ref, g_ref, be_ref,
               bt_ref, wf1_ref, bf1_ref, wf2_ref, bf2_ref, o_ref,
               st_acc, pool_acc, cnt_acc):
    i = pl.program_id(0)
    dinv = _dinv_of(dA_ref, dB_ref)
    z = _z_of(a0_ref, a1_ref, dinv, b_ref)
    _accum_stats(i, z, st_acc)

    @pl.when(i >= NBLK)
    def _():
        y = _bn_relu(z, g_ref, be_ref, st_acc[...])
        bt = jnp.broadcast_to(bt_ref[...][0], (G, BR))
        oh = (bt == lax.broadcasted_iota(jnp.int32, (G, BR), 0)).astype(
            jnp.float32)
        pool_part = _dot(oh, y)
        cnt_part = jnp.broadcast_to(
            jnp.sum(oh, axis=1, keepdims=True), (G, DEGS))

        @pl.when(i == NBLK)
        def _():
            pool_acc[...] = pool_part
            cnt_acc[...] = cnt_part

        @pl.when(i > NBLK)
        def _():
            pool_acc[...] += pool_part
            cnt_acc[...] += cnt_part

        @pl.when(i == 2 * NBLK - 1)
        def _():
            p = pool_acc[...] / jnp.maximum(cnt_acc[:, :1], 1.0)
            q = jnp.maximum(_dot(p, wf1_ref[...]) + bf1_ref[...], 0.0)
            o_ref[...] = _dot(q, wf2_ref[...]) + bf2_ref[...]


def _head(a0, a1, dA, dB, b, g, be, bt3, Wf1, bf1, Wf2, bf2):
    return pl.pallas_call(
        _head_body,
        grid=(2 * NBLK,),
        in_specs=[
            pl.BlockSpec((BR, HALF), lambda i: (i % NBLK, 0)),
            pl.BlockSpec((BR, HALF), lambda i: (i % NBLK, 0)),
            pl.BlockSpec((BR, DEGS), lambda i: (i % NBLK, 0)),
            pl.BlockSpec((BR, DEGS), lambda i: (i % NBLK, 0)),
            pl.BlockSpec((1, D), lambda i: (0, 0)),
            pl.BlockSpec((1, D), lambda i: (0, 0)),
            pl.BlockSpec((1, D), lambda i: (0, 0)),
            pl.BlockSpec((1, 1, BR), lambda i: (i % NBLK, 0, 0)),
            pl.BlockSpec((D, D), lambda i: (0, 0)),
            pl.BlockSpec((1, D), lambda i: (0, 0)),
            pl.BlockSpec((D, 1), lambda i: (0, 0)),
            pl.BlockSpec((1, 1), lambda i: (0, 0)),
        ],
        out_specs=pl.BlockSpec((G, 1), lambda i: (0, 0)),
        out_shape=jax.ShapeDtypeStruct((G, 1), jnp.float32),
        scratch_shapes=[
            pltpu.VMEM((8, D), jnp.float32),
            pltpu.VMEM((G, D), jnp.float32),
            pltpu.VMEM((G, DEGS), jnp.float32),
        ],
    )(a0, a1, dA, dB, b, g, be, bt3, Wf1, bf1, Wf2, bf2)


def kernel(x, edge_index, batch, W1, b1, g1, be1, W2, b2, g2, be2,
           W3, b3, g3, be3, Wf1, bf1, Wf2, bf2):
    src2d = edge_index[0].reshape(ROWS_E, CHUNK)
    dst2d = edge_index[1].reshape(ROWS_E, CHUNK)
    zeros_nd = jnp.zeros((N, DEGW), jnp.float32)
    ones_cd = jnp.ones((CHUNK, DEGW), jnp.float32)
    bt3 = batch.reshape(NBLK, 1, BR)
    b1r, g1r, be1r = b1.reshape(1, D), g1.reshape(1, D), be1.reshape(1, D)
    b2r, g2r, be2r = b2.reshape(1, D), g2.reshape(1, D), be2.reshape(1, D)
    b3r, g3r, be3r = b3.reshape(1, D), g3.reshape(1, D), be3.reshape(1, D)

    dA, dB = _deg_kernel(dst2d, zeros_nd, ones_cd)
    dA, dB = dA[:, :DEGS], dB[:, :DEGS]

    hs0, hs1 = _mm1(x, W1, dA, dB)
    a0, a1 = _agg_kernel(hs0, hs1, src2d, dst2d)

    hs0, hs1 = _mmn(a0, a1, dA, dB, b1r, g1r, be1r, W2)
    a0, a1 = _agg_kernel(hs0, hs1, src2d, dst2d)

    hs0, hs1 = _mmn(a0, a1, dA, dB, b2r, g2r, be2r, W3)
    a0, a1 = _agg_kernel(hs0, hs1, src2d, dst2d)

    return _head(a0, a1, dA, dB, b3r, g3r, be3r, bt3,
                 Wf1, bf1.reshape(1, D), Wf2, bf2.reshape(1, 1))


# speedup vs baseline: 1.0943x; 1.0330x over previous
import functools

import jax
import jax.numpy as jnp
from jax import lax
from jax.experimental import pallas as pl
from jax.experimental.pallas import tpu as pltpu
from jax.experimental.pallas import tpu_sc as plsc

N = 10000
E = 160000
D = 256
G = 64
EPS = 1e-5

NC = 2
NS = 16
CHUNK = 125
GRP = 40
ROWS_E = E // CHUNK
RPT_AGG = ROWS_E // NS
RPT_DEG = ROWS_E // (NC * NS)
NPT = 624
TAIL0 = NS * NPT
TAILN = N - TAIL0
DEGW = 128
DEGS = 8
HALF = D // 2
BR = 1000
NBLK = N // BR

_sc_mesh = plsc.VectorSubcoreMesh(
    core_axis_name="c", subcore_axis_name="s", num_cores=NC, num_subcores=NS)


def _part_copy(src, dst, s):
    r0 = s * NPT
    pltpu.sync_copy(src.at[pl.ds(r0, NPT)], dst.at[pl.ds(r0, NPT)])

    @pl.when(s == NS - 1)
    def _():
        pltpu.sync_copy(src.at[pl.ds(TAIL0, TAILN)], dst.at[pl.ds(TAIL0, TAILN)])


@functools.partial(
    pl.kernel,
    out_type=(jax.ShapeDtypeStruct((N, DEGW), jnp.float32),
              jax.ShapeDtypeStruct((N, DEGW), jnp.float32)),
    mesh=_sc_mesh,
    scratch_types=[
        pltpu.VMEM((RPT_DEG, CHUNK), jnp.int32),
        pltpu.VMEM((CHUNK, DEGW), jnp.float32),
        pltpu.VMEM_SHARED((N, DEGW), jnp.float32),
    ],
)
def _deg_kernel(dst_hbm, zeros_hbm, ones_hbm, out0, out1, idx_v, ones_v, acc):
    c = lax.axis_index("c")
    s = lax.axis_index("s")
    w = s * NC + c
    _part_copy(zeros_hbm, acc, s)
    pltpu.sync_copy(ones_hbm, ones_v)
    pltpu.sync_copy(dst_hbm.at[pl.ds(w * RPT_DEG, RPT_DEG)], idx_v)
    plsc.subcore_barrier()

    def body(j, carry):
        pltpu.sync_copy(ones_v, acc.at[idx_v.at[j]], add=True)
        return carry

    lax.fori_loop(0, RPT_DEG, body, 0)
    plsc.subcore_barrier()

    @pl.when(c == 0)
    def _():
        _part_copy(acc, out0, s)

    @pl.when(c == 1)
    def _():
        _part_copy(acc, out1, s)


@functools.partial(
    pl.kernel,
    out_type=(jax.ShapeDtypeStruct((N, HALF), jnp.float32),
              jax.ShapeDtypeStruct((N, HALF), jnp.float32)),
    mesh=_sc_mesh,
    scratch_types=[
        pltpu.VMEM((GRP, CHUNK), jnp.int32),
        pltpu.VMEM((GRP, CHUNK), jnp.int32),
        pltpu.VMEM((2, CHUNK, HALF), jnp.float32),
        pltpu.VMEM_SHARED((N, HALF), jnp.float32),
        pltpu.SemaphoreType.DMA,
        pltpu.SemaphoreType.DMA,
    ],
)
def _agg_kernel(hs0, hs1, src_hbm, dst_hbm, out0, out1,
                src_v, dst_v, rows_v, acc, sem0, sem1):
    c = lax.axis_index("c")
    s = lax.axis_index("s")
    e0 = s * RPT_AGG

    @pl.when(c == 0)
    def _():
        _part_copy(hs0, acc, s)

    @pl.when(c == 1)
    def _():
        _part_copy(hs1, acc, s)

    plsc.subcore_barrier()

    def _pipe(hs_ref):
        def group(g, carry):
            pltpu.sync_copy(src_hbm.at[pl.ds(e0 + g * GRP, GRP)], src_v)
            pltpu.sync_copy(dst_hbm.at[pl.ds(e0 + g * GRP, GRP)], dst_v)
            pltpu.async_copy(hs_ref.at[src_v.at[0]], rows_v.at[0], sem0)

            def pair(p, carry2):
                j = 2 * p
                pltpu.make_async_copy(
                    hs_ref.at[src_v.at[j]], rows_v.at[0], sem0).wait()
                pltpu.async_copy(hs_ref.at[src_v.at[j + 1]], rows_v.at[1], sem1)
                pltpu.sync_copy(rows_v.at[0], acc.at[dst_v.at[j]], add=True)

                pltpu.make_async_copy(
                    hs_ref.at[src_v.at[j + 1]], rows_v.at[1], sem1).wait()

                @pl.when(p < GRP // 2 - 1)
                def _():
                    pltpu.async_copy(
                        hs_ref.at[src_v.at[j + 2]], rows_v.at[0], sem0)

                pltpu.sync_copy(rows_v.at[1], acc.at[dst_v.at[j + 1]], add=True)
                return carry2

            lax.fori_loop(0, GRP // 2, pair, 0)
            return carry

        lax.fori_loop(0, RPT_AGG // GRP, group, 0)

    @pl.when(c == 0)
    def _():
        _pipe(hs0)

    @pl.when(c == 1)
    def _():
        _pipe(hs1)

    plsc.subcore_barrier()

    @pl.when(c == 0)
    def _():
        _part_copy(acc, out0, s)

    @pl.when(c == 1)
    def _():
        _part_copy(acc, out1, s)


def _dot(a, b):
    return jnp.dot(a, b, preferred_element_type=jnp.float32)


def _dinv_of(dA_ref, dB_ref):
    deg = dA_ref[:, :1] + dB_ref[:, :1] + 1.0
    return lax.rsqrt(deg)


def _mm1_body(x_ref, w_ref, dA_ref, dB_ref, o0_ref, o1_ref):
    dinv = _dinv_of(dA_ref, dB_ref)
    h = _dot(x_ref[...], w_ref[...])
    hs = h * dinv
    o0_ref[...] = hs[:, :HALF]
    o1_ref[...] = hs[:, HALF:]


def _mm1(x, W, dA, dB):
    return pl.pallas_call(
        _mm1_body,
        grid=(NBLK,),
        in_specs=[
            pl.BlockSpec((BR, D), lambda i: (i, 0)),
            pl.BlockSpec((D, D), lambda i: (0, 0)),
            pl.BlockSpec((BR, DEGS), lambda i: (i, 0)),
            pl.BlockSpec((BR, DEGS), lambda i: (i, 0)),
        ],
        out_specs=[pl.BlockSpec((BR, HALF), lambda i: (i, 0))] * 2,
        out_shape=[jax.ShapeDtypeStruct((N, HALF), jnp.float32)] * 2,
    )(x, W, dA, dB)


def _z_of(a0_ref, a1_ref, dinv, b_ref):
    z = jnp.concatenate([a0_ref[...], a1_ref[...]], axis=1)
    return z * dinv + b_ref[...]


def _stats_part(z):
    ps = jnp.sum(z, axis=0, keepdims=True)
    pss = jnp.sum(z * z, axis=0, keepdims=True)
    return jnp.concatenate([ps, pss, jnp.zeros((6, D), jnp.float32)], axis=0)


def _accum_stats(i, z, st_acc):
    @pl.when(i == 0)
    def _():
        st_acc[...] = _stats_part(z)

    @pl.when((i > 0) & (i < NBLK))
    def _():
        st_acc[...] += _stats_part(z)


def _bn_relu(z, g_ref, be_ref, st):
    m = st[0:1, :] * (1.0 / N)
    v = st[1:2, :] * (1.0 / N) - m * m
    return jnp.maximum(g_ref[...] * (z - m) * lax.rsqrt(v + EPS) + be_ref[...], 0.0)


def _mmn_body(a0_ref, a1_ref, dA_ref, dB_ref, b_ref, g_ref, be_ref,
              w_ref, o0_ref, o1_ref, st_acc):
    i = pl.program_id(0)
    dinv = _dinv_of(dA_ref, dB_ref)
    z = _z_of(a0_ref, a1_ref, dinv, b_ref)
    _accum_stats(i, z, st_acc)

    @pl.when(i >= NBLK)
    def _():
        y = _bn_relu(z, g_ref, be_ref, st_acc[...])
        h = _dot(y, w_ref[...])
        hs = h * dinv
        o0_ref[...] = hs[:, :HALF]
        o1_ref[...] = hs[:, HALF:]


def _mmn(a0, a1, dA, dB, b, g, be, W):
    return pl.pallas_call(
        _mmn_body,
        grid=(2 * NBLK,),
        in_specs=[
            pl.BlockSpec((BR, HALF), lambda i: (i % NBLK, 0)),
            pl.BlockSpec((BR, HALF), lambda i: (i % NBLK, 0)),
            pl.BlockSpec((BR, DEGS), lambda i: (i % NBLK, 0)),
            pl.BlockSpec((BR, DEGS), lambda i: (i % NBLK, 0)),
            pl.BlockSpec((1, D), lambda i: (0, 0)),
            pl.BlockSpec((1, D), lambda i: (0, 0)),
            pl.BlockSpec((1, D), lambda i: (0, 0)),
            pl.BlockSpec((D, D), lambda i: (0, 0)),
        ],
        out_specs=[pl.BlockSpec((BR, HALF), lambda i: (i % NBLK, 0))] * 2,
        out_shape=[jax.ShapeDtypeStruct((N, HALF), jnp.float32)] * 2,
        scratch_shapes=[pltpu.VMEM((8, D), jnp.float32)],
    )(a0, a1, dA, dB, b, g, be, W)


def _head_body(a0_ref, a1_ref, dA_ref, dB_ref, b_ref, g_ref, be_ref,
               bt_ref, wf1_ref, bf1_ref, wf2_ref, bf2_ref, o_ref,
               st_acc, pool_acc, cnt_acc):
    i = pl.program_id(0)
    dinv = _dinv_of(dA_ref, dB_ref)
    z = _z_of(a0_ref, a1_ref, dinv, b_ref)
    _accum_stats(i, z, st_acc)

    @pl.when(i >= NBLK)
    def _():
        y = _bn_relu(z, g_ref, be_ref, st_acc[...])
        bt = jnp.broadcast_to(bt_ref[...][0], (G, BR))
        oh = (bt == lax.broadcasted_iota(jnp.int32, (G, BR), 0)).astype(
            jnp.float32)
        pool_part = _dot(oh, y)
        cnt_part = jnp.broadcast_to(
            jnp.sum(oh, axis=1, keepdims=True), (G, DEGS))

        @pl.when(i == NBLK)
        def _():
            pool_acc[...] = pool_part
            cnt_acc[...] = cnt_part

        @pl.when(i > NBLK)
        def _():
            pool_acc[...] += pool_part
            cnt_acc[...] += cnt_part

        @pl.when(i == 2 * NBLK - 1)
        def _():
            p = pool_acc[...] / jnp.maximum(cnt_acc[:, :1], 1.0)
            q = jnp.maximum(_dot(p, wf1_ref[...]) + bf1_ref[...], 0.0)
            o_ref[...] = _dot(q, wf2_ref[...]) + bf2_ref[...]


def _head(a0, a1, dA, dB, b, g, be, bt3, Wf1, bf1, Wf2, bf2):
    return pl.pallas_call(
        _head_body,
        grid=(2 * NBLK,),
        in_specs=[
            pl.BlockSpec((BR, HALF), lambda i: (i % NBLK, 0)),
            pl.BlockSpec((BR, HALF), lambda i: (i % NBLK, 0)),
            pl.BlockSpec((BR, DEGS), lambda i: (i % NBLK, 0)),
            pl.BlockSpec((BR, DEGS), lambda i: (i % NBLK, 0)),
            pl.BlockSpec((1, D), lambda i: (0, 0)),
            pl.BlockSpec((1, D), lambda i: (0, 0)),
            pl.BlockSpec((1, D), lambda i: (0, 0)),
            pl.BlockSpec((1, 1, BR), lambda i: (i % NBLK, 0, 0)),
            pl.BlockSpec((D, D), lambda i: (0, 0)),
            pl.BlockSpec((1, D), lambda i: (0, 0)),
            pl.BlockSpec((D, 1), lambda i: (0, 0)),
            pl.BlockSpec((1, 1), lambda i: (0, 0)),
        ],
        out_specs=pl.BlockSpec((G, 1), lambda i: (0, 0)),
        out_shape=jax.ShapeDtypeStruct((G, 1), jnp.float32),
        scratch_shapes=[
            pltpu.VMEM((8, D), jnp.float32),
            pltpu.VMEM((G, D), jnp.float32),
            pltpu.VMEM((G, DEGS), jnp.float32),
        ],
    )(a0, a1, dA, dB, b, g, be, bt3, Wf1, bf1, Wf2, bf2)


def kernel(x, edge_index, batch, W1, b1, g1, be1, W2, b2, g2, be2,
           W3, b3, g3, be3, Wf1, bf1, Wf2, bf2):
    src2d = edge_index[0].reshape(ROWS_E, CHUNK)
    dst2d = edge_index[1].reshape(ROWS_E, CHUNK)
    zeros_nd = jnp.zeros((N, DEGW), jnp.float32)
    ones_cd = jnp.ones((CHUNK, DEGW), jnp.float32)
    bt3 = batch.reshape(NBLK, 1, BR)
    b1r, g1r, be1r = b1.reshape(1, D), g1.reshape(1, D), be1.reshape(1, D)
    b2r, g2r, be2r = b2.reshape(1, D), g2.reshape(1, D), be2.reshape(1, D)
    b3r, g3r, be3r = b3.reshape(1, D), g3.reshape(1, D), be3.reshape(1, D)

    dA, dB = _deg_kernel(dst2d, zeros_nd, ones_cd)
    dA, dB = dA[:, :DEGS], dB[:, :DEGS]

    hs0, hs1 = _mm1(x, W1, dA, dB)
    a0, a1 = _agg_kernel(hs0, hs1, src2d, dst2d)

    hs0, hs1 = _mmn(a0, a1, dA, dB, b1r, g1r, be1r, W2)
    a0, a1 = _agg_kernel(hs0, hs1, src2d, dst2d)

    hs0, hs1 = _mmn(a0, a1, dA, dB, b2r, g2r, be2r, W3)
    a0, a1 = _agg_kernel(hs0, hs1, src2d, dst2d)

    return _head(a0, a1, dA, dB, b3r, g3r, be3r, bt3,
                 Wf1, bf1.reshape(1, D), Wf2, bf2.reshape(1, 1))


# speedup vs baseline: 1.1010x; 1.0061x over previous
import functools

import jax
import jax.numpy as jnp
from jax import lax
from jax.experimental import pallas as pl
from jax.experimental.pallas import tpu as pltpu
from jax.experimental.pallas import tpu_sc as plsc

N = 10000
E = 160000
D = 256
G = 64
EPS = 1e-5

NC = 2
NS = 16
CHUNK = 125
GRP = 40
ROWS_E = E // CHUNK
RPT_AGG = ROWS_E // NS
RPT_DEG = ROWS_E // (NC * NS)
NPT = 624
TAIL0 = NS * NPT
TAILN = N - TAIL0
DEGW = 128
DEGS = 8
HALF = D // 2
BR = 1000
NBLK = N // BR

_sc_mesh = plsc.VectorSubcoreMesh(
    core_axis_name="c", subcore_axis_name="s", num_cores=NC, num_subcores=NS)


def _part_copy(src, dst, s):
    r0 = s * NPT
    pltpu.sync_copy(src.at[pl.ds(r0, NPT)], dst.at[pl.ds(r0, NPT)])

    @pl.when(s == NS - 1)
    def _():
        pltpu.sync_copy(src.at[pl.ds(TAIL0, TAILN)], dst.at[pl.ds(TAIL0, TAILN)])


@functools.partial(
    pl.kernel,
    out_type=(jax.ShapeDtypeStruct((N, DEGW), jnp.float32),
              jax.ShapeDtypeStruct((N, DEGW), jnp.float32)),
    mesh=_sc_mesh,
    scratch_types=[
        pltpu.VMEM((RPT_DEG, CHUNK), jnp.int32),
        pltpu.VMEM((CHUNK, DEGW), jnp.float32),
        pltpu.VMEM_SHARED((N, DEGW), jnp.float32),
    ],
)
def _deg_kernel(dst_hbm, zeros_hbm, ones_hbm, out0, out1, idx_v, ones_v, acc):
    c = lax.axis_index("c")
    s = lax.axis_index("s")
    w = s * NC + c
    _part_copy(zeros_hbm, acc, s)
    pltpu.sync_copy(ones_hbm, ones_v)
    pltpu.sync_copy(dst_hbm.at[pl.ds(w * RPT_DEG, RPT_DEG)], idx_v)
    plsc.subcore_barrier()

    def body(j, carry):
        pltpu.sync_copy(ones_v, acc.at[idx_v.at[j]], add=True)
        return carry

    lax.fori_loop(0, RPT_DEG, body, 0)
    plsc.subcore_barrier()

    @pl.when(c == 0)
    def _():
        _part_copy(acc, out0, s)

    @pl.when(c == 1)
    def _():
        _part_copy(acc, out1, s)


@functools.partial(
    pl.kernel,
    out_type=(jax.ShapeDtypeStruct((N, HALF), jnp.float32),
              jax.ShapeDtypeStruct((N, HALF), jnp.float32)),
    mesh=_sc_mesh,
    scratch_types=[
        pltpu.VMEM((GRP, CHUNK), jnp.int32),
        pltpu.VMEM((GRP, CHUNK), jnp.int32),
        pltpu.VMEM((2, CHUNK, HALF), jnp.float32),
        pltpu.VMEM_SHARED((N, HALF), jnp.float32),
        pltpu.SemaphoreType.DMA,
        pltpu.SemaphoreType.DMA,
    ],
)
def _agg_kernel(hs0, hs1, src_hbm, dst_hbm, out0, out1,
                src_v, dst_v, rows_v, acc, sem0, sem1):
    c = lax.axis_index("c")
    s = lax.axis_index("s")
    e0 = s * RPT_AGG

    def _start(hs_ref):
        pltpu.sync_copy(src_hbm.at[pl.ds(e0, GRP)], src_v)
        pltpu.sync_copy(dst_hbm.at[pl.ds(e0, GRP)], dst_v)
        pltpu.async_copy(hs_ref.at[src_v.at[0]], rows_v.at[0], sem0)

    @pl.when(c == 0)
    def _():
        _start(hs0)
        _part_copy(hs0, acc, s)

    @pl.when(c == 1)
    def _():
        _start(hs1)
        _part_copy(hs1, acc, s)

    plsc.subcore_barrier()

    def _pipe(hs_ref):
        def group(g, carry):
            @pl.when(g > 0)
            def _():
                pltpu.sync_copy(src_hbm.at[pl.ds(e0 + g * GRP, GRP)], src_v)
                pltpu.sync_copy(dst_hbm.at[pl.ds(e0 + g * GRP, GRP)], dst_v)
                pltpu.async_copy(hs_ref.at[src_v.at[0]], rows_v.at[0], sem0)

            def pair(p, carry2):
                j = 2 * p
                pltpu.make_async_copy(
                    hs_ref.at[src_v.at[j]], rows_v.at[0], sem0).wait()
                pltpu.async_copy(hs_ref.at[src_v.at[j + 1]], rows_v.at[1], sem1)
                pltpu.sync_copy(rows_v.at[0], acc.at[dst_v.at[j]], add=True)

                pltpu.make_async_copy(
                    hs_ref.at[src_v.at[j + 1]], rows_v.at[1], sem1).wait()

                @pl.when(p < GRP // 2 - 1)
                def _():
                    pltpu.async_copy(
                        hs_ref.at[src_v.at[j + 2]], rows_v.at[0], sem0)

                pltpu.sync_copy(rows_v.at[1], acc.at[dst_v.at[j + 1]], add=True)
                return carry2

            lax.fori_loop(0, GRP // 2, pair, 0)
            return carry

        lax.fori_loop(0, RPT_AGG // GRP, group, 0)

    @pl.when(c == 0)
    def _():
        _pipe(hs0)

    @pl.when(c == 1)
    def _():
        _pipe(hs1)

    plsc.subcore_barrier()

    @pl.when(c == 0)
    def _():
        _part_copy(acc, out0, s)

    @pl.when(c == 1)
    def _():
        _part_copy(acc, out1, s)


def _dot(a, b):
    return jnp.dot(a, b, preferred_element_type=jnp.float32)


def _dinv_of(dA_ref, dB_ref):
    deg = dA_ref[:, :1] + dB_ref[:, :1] + 1.0
    return lax.rsqrt(deg)


def _mm1_body(x_ref, w_ref, dA_ref, dB_ref, o0_ref, o1_ref):
    dinv = _dinv_of(dA_ref, dB_ref)
    h = _dot(x_ref[...], w_ref[...])
    hs = h * dinv
    o0_ref[...] = hs[:, :HALF]
    o1_ref[...] = hs[:, HALF:]


def _mm1(x, W, dA, dB):
    return pl.pallas_call(
        _mm1_body,
        grid=(NBLK,),
        in_specs=[
            pl.BlockSpec((BR, D), lambda i: (i, 0)),
            pl.BlockSpec((D, D), lambda i: (0, 0)),
            pl.BlockSpec((BR, DEGS), lambda i: (i, 0)),
            pl.BlockSpec((BR, DEGS), lambda i: (i, 0)),
        ],
        out_specs=[pl.BlockSpec((BR, HALF), lambda i: (i, 0))] * 2,
        out_shape=[jax.ShapeDtypeStruct((N, HALF), jnp.float32)] * 2,
    )(x, W, dA, dB)


def _z_of(a0_ref, a1_ref, dinv, b_ref):
    z = jnp.concatenate([a0_ref[...], a1_ref[...]], axis=1)
    return z * dinv + b_ref[...]


def _stats_part(z):
    ps = jnp.sum(z, axis=0, keepdims=True)
    pss = jnp.sum(z * z, axis=0, keepdims=True)
    return jnp.concatenate([ps, pss, jnp.zeros((6, D), jnp.float32)], axis=0)


def _accum_stats(i, z, st_acc):
    @pl.when(i == 0)
    def _():
        st_acc[...] = _stats_part(z)

    @pl.when((i > 0) & (i < NBLK))
    def _():
        st_acc[...] += _stats_part(z)


def _bn_relu(z, g_ref, be_ref, st):
    m = st[0:1, :] * (1.0 / N)
    v = st[1:2, :] * (1.0 / N) - m * m
    return jnp.maximum(g_ref[...] * (z - m) * lax.rsqrt(v + EPS) + be_ref[...], 0.0)


def _mmn_body(a0_ref, a1_ref, dA_ref, dB_ref, b_ref, g_ref, be_ref,
              w_ref, o0_ref, o1_ref, st_acc):
    i = pl.program_id(0)
    dinv = _dinv_of(dA_ref, dB_ref)
    z = _z_of(a0_ref, a1_ref, dinv, b_ref)
    _accum_stats(i, z, st_acc)

    @pl.when(i >= NBLK)
    def _():
        y = _bn_relu(z, g_ref, be_ref, st_acc[...])
        h = _dot(y, w_ref[...])
        hs = h * dinv
        o0_ref[...] = hs[:, :HALF]
        o1_ref[...] = hs[:, HALF:]


def _mmn(a0, a1, dA, dB, b, g, be, W):
    return pl.pallas_call(
        _mmn_body,
        grid=(2 * NBLK,),
        in_specs=[
            pl.BlockSpec((BR, HALF), lambda i: (i % NBLK, 0)),
            pl.BlockSpec((BR, HALF), lambda i: (i % NBLK, 0)),
            pl.BlockSpec((BR, DEGS), lambda i: (i % NBLK, 0)),
            pl.BlockSpec((BR, DEGS), lambda i: (i % NBLK, 0)),
            pl.BlockSpec((1, D), lambda i: (0, 0)),
            pl.BlockSpec((1, D), lambda i: (0, 0)),
            pl.BlockSpec((1, D), lambda i: (0, 0)),
            pl.BlockSpec((D, D), lambda i: (0, 0)),
        ],
        out_specs=[pl.BlockSpec((BR, HALF), lambda i: (i % NBLK, 0))] * 2,
        out_shape=[jax.ShapeDtypeStruct((N, HALF), jnp.float32)] * 2,
        scratch_shapes=[pltpu.VMEM((8, D), jnp.float32)],
    )(a0, a1, dA, dB, b, g, be, W)


def _head_body(a0_ref, a1_ref, dA_ref, dB_ref, b_ref, g_ref, be_ref,
               bt_ref, wf1_ref, bf1_ref, wf2_ref, bf2_ref, o_ref,
               st_acc, pool_acc, cnt_acc):
    i = pl.program_id(0)
    dinv = _dinv_of(dA_ref, dB_ref)
    z = _z_of(a0_ref, a1_ref, dinv, b_ref)
    _accum_stats(i, z, st_acc)

    @pl.when(i >= NBLK)
    def _():
        y = _bn_relu(z, g_ref, be_ref, st_acc[...])
        bt = jnp.broadcast_to(bt_ref[...][0], (G, BR))
        oh = (bt == lax.broadcasted_iota(jnp.int32, (G, BR), 0)).astype(
            jnp.float32)
        pool_part = _dot(oh, y)
        cnt_part = jnp.broadcast_to(
            jnp.sum(oh, axis=1, keepdims=True), (G, DEGS))

        @pl.when(i == NBLK)
        def _():
            pool_acc[...] = pool_part
            cnt_acc[...] = cnt_part

        @pl.when(i > NBLK)
        def _():
            pool_acc[...] += pool_part
            cnt_acc[...] += cnt_part

        @pl.when(i == 2 * NBLK - 1)
        def _():
            p = pool_acc[...] / jnp.maximum(cnt_acc[:, :1], 1.0)
            q = jnp.maximum(_dot(p, wf1_ref[...]) + bf1_ref[...], 0.0)
            o_ref[...] = _dot(q, wf2_ref[...]) + bf2_ref[...]


def _head(a0, a1, dA, dB, b, g, be, bt3, Wf1, bf1, Wf2, bf2):
    return pl.pallas_call(
        _head_body,
        grid=(2 * NBLK,),
        in_specs=[
            pl.BlockSpec((BR, HALF), lambda i: (i % NBLK, 0)),
            pl.BlockSpec((BR, HALF), lambda i: (i % NBLK, 0)),
            pl.BlockSpec((BR, DEGS), lambda i: (i % NBLK, 0)),
            pl.BlockSpec((BR, DEGS), lambda i: (i % NBLK, 0)),
            pl.BlockSpec((1, D), lambda i: (0, 0)),
            pl.BlockSpec((1, D), lambda i: (0, 0)),
            pl.BlockSpec((1, D), lambda i: (0, 0)),
            pl.BlockSpec((1, 1, BR), lambda i: (i % NBLK, 0, 0)),
            pl.BlockSpec((D, D), lambda i: (0, 0)),
            pl.BlockSpec((1, D), lambda i: (0, 0)),
            pl.BlockSpec((D, 1), lambda i: (0, 0)),
            pl.BlockSpec((1, 1), lambda i: (0, 0)),
        ],
        out_specs=pl.BlockSpec((G, 1), lambda i: (0, 0)),
        out_shape=jax.ShapeDtypeStruct((G, 1), jnp.float32),
        scratch_shapes=[
            pltpu.VMEM((8, D), jnp.float32),
            pltpu.VMEM((G, D), jnp.float32),
            pltpu.VMEM((G, DEGS), jnp.float32),
        ],
    )(a0, a1, dA, dB, b, g, be, bt3, Wf1, bf1, Wf2, bf2)


def kernel(x, edge_index, batch, W1, b1, g1, be1, W2, b2, g2, be2,
           W3, b3, g3, be3, Wf1, bf1, Wf2, bf2):
    src2d = edge_index[0].reshape(ROWS_E, CHUNK)
    dst2d = edge_index[1].reshape(ROWS_E, CHUNK)
    zeros_nd = jnp.zeros((N, DEGW), jnp.float32)
    ones_cd = jnp.ones((CHUNK, DEGW), jnp.float32)
    bt3 = batch.reshape(NBLK, 1, BR)
    b1r, g1r, be1r = b1.reshape(1, D), g1.reshape(1, D), be1.reshape(1, D)
    b2r, g2r, be2r = b2.reshape(1, D), g2.reshape(1, D), be2.reshape(1, D)
    b3r, g3r, be3r = b3.reshape(1, D), g3.reshape(1, D), be3.reshape(1, D)

    dA, dB = _deg_kernel(dst2d, zeros_nd, ones_cd)
    dA, dB = dA[:, :DEGS], dB[:, :DEGS]

    hs0, hs1 = _mm1(x, W1, dA, dB)
    a0, a1 = _agg_kernel(hs0, hs1, src2d, dst2d)

    hs0, hs1 = _mmn(a0, a1, dA, dB, b1r, g1r, be1r, W2)
    a0, a1 = _agg_kernel(hs0, hs1, src2d, dst2d)

    hs0, hs1 = _mmn(a0, a1, dA, dB, b2r, g2r, be2r, W3)
    a0, a1 = _agg_kernel(hs0, hs1, src2d, dst2d)

    return _head(a0, a1, dA, dB, b3r, g3r, be3r, bt3,
                 Wf1, bf1.reshape(1, D), Wf2, bf2.reshape(1, 1))


# speedup vs baseline: 1.1308x; 1.0271x over previous
import functools

import jax
import jax.numpy as jnp
from jax import lax
from jax.experimental import pallas as pl
from jax.experimental.pallas import tpu as pltpu
from jax.experimental.pallas import tpu_sc as plsc

N = 10000
E = 160000
D = 256
G = 64
EPS = 1e-5

NC = 2
NS = 16
CHUNK = 125
GRP = 40
ROWS_E = E // CHUNK
RPT_AGG = ROWS_E // NS
RPT_DEG = ROWS_E // (NC * NS)
NPT = 624
TAIL0 = NS * NPT
TAILN = N - TAIL0
DEGW = 128
DEGS = 8
HALF = D // 2
BR = 2000
NBLK = N // BR

_sc_mesh = plsc.VectorSubcoreMesh(
    core_axis_name="c", subcore_axis_name="s", num_cores=NC, num_subcores=NS)


def _part_copy(src, dst, s):
    r0 = s * NPT
    pltpu.sync_copy(src.at[pl.ds(r0, NPT)], dst.at[pl.ds(r0, NPT)])

    @pl.when(s == NS - 1)
    def _():
        pltpu.sync_copy(src.at[pl.ds(TAIL0, TAILN)], dst.at[pl.ds(TAIL0, TAILN)])


@functools.partial(
    pl.kernel,
    out_type=(jax.ShapeDtypeStruct((N, DEGW), jnp.float32),
              jax.ShapeDtypeStruct((N, DEGW), jnp.float32)),
    mesh=_sc_mesh,
    scratch_types=[
        pltpu.VMEM((RPT_DEG, CHUNK), jnp.int32),
        pltpu.VMEM((CHUNK, DEGW), jnp.float32),
        pltpu.VMEM_SHARED((N, DEGW), jnp.float32),
    ],
)
def _deg_kernel(dst_hbm, zeros_hbm, ones_hbm, out0, out1, idx_v, ones_v, acc):
    c = lax.axis_index("c")
    s = lax.axis_index("s")
    w = s * NC + c
    _part_copy(zeros_hbm, acc, s)
    pltpu.sync_copy(ones_hbm, ones_v)
    pltpu.sync_copy(dst_hbm.at[pl.ds(w * RPT_DEG, RPT_DEG)], idx_v)
    plsc.subcore_barrier()

    def body(j, carry):
        pltpu.sync_copy(ones_v, acc.at[idx_v.at[j]], add=True)
        return carry

    lax.fori_loop(0, RPT_DEG, body, 0)
    plsc.subcore_barrier()

    @pl.when(c == 0)
    def _():
        _part_copy(acc, out0, s)

    @pl.when(c == 1)
    def _():
        _part_copy(acc, out1, s)


@functools.partial(
    pl.kernel,
    out_type=(jax.ShapeDtypeStruct((N, HALF), jnp.float32),
              jax.ShapeDtypeStruct((N, HALF), jnp.float32)),
    mesh=_sc_mesh,
    scratch_types=[
        pltpu.VMEM((GRP, CHUNK), jnp.int32),
        pltpu.VMEM((GRP, CHUNK), jnp.int32),
        pltpu.VMEM((2, CHUNK, HALF), jnp.float32),
        pltpu.VMEM_SHARED((N, HALF), jnp.float32),
        pltpu.SemaphoreType.DMA,
        pltpu.SemaphoreType.DMA,
    ],
)
def _agg_kernel(hs0, hs1, src_hbm, dst_hbm, out0, out1,
                src_v, dst_v, rows_v, acc, sem0, sem1):
    c = lax.axis_index("c")
    s = lax.axis_index("s")
    e0 = s * RPT_AGG

    def _start(hs_ref):
        pltpu.sync_copy(src_hbm.at[pl.ds(e0, GRP)], src_v)
        pltpu.sync_copy(dst_hbm.at[pl.ds(e0, GRP)], dst_v)
        pltpu.async_copy(hs_ref.at[src_v.at[0]], rows_v.at[0], sem0)

    @pl.when(c == 0)
    def _():
        _start(hs0)
        _part_copy(hs0, acc, s)

    @pl.when(c == 1)
    def _():
        _start(hs1)
        _part_copy(hs1, acc, s)

    plsc.subcore_barrier()

    def _pipe(hs_ref):
        def group(g, carry):
            @pl.when(g > 0)
            def _():
                pltpu.sync_copy(src_hbm.at[pl.ds(e0 + g * GRP, GRP)], src_v)
                pltpu.sync_copy(dst_hbm.at[pl.ds(e0 + g * GRP, GRP)], dst_v)
                pltpu.async_copy(hs_ref.at[src_v.at[0]], rows_v.at[0], sem0)

            def pair(p, carry2):
                j = 2 * p
                pltpu.make_async_copy(
                    hs_ref.at[src_v.at[j]], rows_v.at[0], sem0).wait()
                pltpu.async_copy(hs_ref.at[src_v.at[j + 1]], rows_v.at[1], sem1)
                pltpu.sync_copy(rows_v.at[0], acc.at[dst_v.at[j]], add=True)

                pltpu.make_async_copy(
                    hs_ref.at[src_v.at[j + 1]], rows_v.at[1], sem1).wait()

                @pl.when(p < GRP // 2 - 1)
                def _():
                    pltpu.async_copy(
                        hs_ref.at[src_v.at[j + 2]], rows_v.at[0], sem0)

                pltpu.sync_copy(rows_v.at[1], acc.at[dst_v.at[j + 1]], add=True)
                return carry2

            lax.fori_loop(0, GRP // 2, pair, 0)
            return carry

        lax.fori_loop(0, RPT_AGG // GRP, group, 0)

    @pl.when(c == 0)
    def _():
        _pipe(hs0)

    @pl.when(c == 1)
    def _():
        _pipe(hs1)

    plsc.subcore_barrier()

    @pl.when(c == 0)
    def _():
        _part_copy(acc, out0, s)

    @pl.when(c == 1)
    def _():
        _part_copy(acc, out1, s)


def _dot(a, b):
    return jnp.dot(a, b, preferred_element_type=jnp.float32)


def _dinv_of(dA_ref, dB_ref):
    deg = dA_ref[:, :1] + dB_ref[:, :1] + 1.0
    return lax.rsqrt(deg)


def _mm1_body(x_ref, w_ref, dA_ref, dB_ref, o0_ref, o1_ref):
    dinv = _dinv_of(dA_ref, dB_ref)
    h = _dot(x_ref[...], w_ref[...])
    hs = h * dinv
    o0_ref[...] = hs[:, :HALF]
    o1_ref[...] = hs[:, HALF:]


def _mm1(x, W, dA, dB):
    return pl.pallas_call(
        _mm1_body,
        grid=(NBLK,),
        in_specs=[
            pl.BlockSpec((BR, D), lambda i: (i, 0)),
            pl.BlockSpec((D, D), lambda i: (0, 0)),
            pl.BlockSpec((BR, DEGS), lambda i: (i, 0)),
            pl.BlockSpec((BR, DEGS), lambda i: (i, 0)),
        ],
        out_specs=[pl.BlockSpec((BR, HALF), lambda i: (i, 0))] * 2,
        out_shape=[jax.ShapeDtypeStruct((N, HALF), jnp.float32)] * 2,
    )(x, W, dA, dB)


def _z_of(a0_ref, a1_ref, dinv, b_ref):
    z = jnp.concatenate([a0_ref[...], a1_ref[...]], axis=1)
    return z * dinv + b_ref[...]


def _stats_part(z):
    ps = jnp.sum(z, axis=0, keepdims=True)
    pss = jnp.sum(z * z, axis=0, keepdims=True)
    return jnp.concatenate([ps, pss, jnp.zeros((6, D), jnp.float32)], axis=0)


def _accum_stats(i, z, st_acc):
    @pl.when(i == 0)
    def _():
        st_acc[...] = _stats_part(z)

    @pl.when((i > 0) & (i < NBLK))
    def _():
        st_acc[...] += _stats_part(z)


def _bn_relu(z, g_ref, be_ref, st):
    m = st[0:1, :] * (1.0 / N)
    v = st[1:2, :] * (1.0 / N) - m * m
    return jnp.maximum(g_ref[...] * (z - m) * lax.rsqrt(v + EPS) + be_ref[...], 0.0)


def _mmn_body(a0_ref, a1_ref, dA_ref, dB_ref, b_ref, g_ref, be_ref,
              w_ref, o0_ref, o1_ref, st_acc):
    i = pl.program_id(0)
    dinv = _dinv_of(dA_ref, dB_ref)
    z = _z_of(a0_ref, a1_ref, dinv, b_ref)
    _accum_stats(i, z, st_acc)

    @pl.when(i >= NBLK)
    def _():
        y = _bn_relu(z, g_ref, be_ref, st_acc[...])
        h = _dot(y, w_ref[...])
        hs = h * dinv
        o0_ref[...] = hs[:, :HALF]
        o1_ref[...] = hs[:, HALF:]


def _mmn(a0, a1, dA, dB, b, g, be, W):
    return pl.pallas_call(
        _mmn_body,
        grid=(2 * NBLK,),
        in_specs=[
            pl.BlockSpec((BR, HALF), lambda i: (i % NBLK, 0)),
            pl.BlockSpec((BR, HALF), lambda i: (i % NBLK, 0)),
            pl.BlockSpec((BR, DEGS), lambda i: (i % NBLK, 0)),
            pl.BlockSpec((BR, DEGS), lambda i: (i % NBLK, 0)),
            pl.BlockSpec((1, D), lambda i: (0, 0)),
            pl.BlockSpec((1, D), lambda i: (0, 0)),
            pl.BlockSpec((1, D), lambda i: (0, 0)),
            pl.BlockSpec((D, D), lambda i: (0, 0)),
        ],
        out_specs=[pl.BlockSpec((BR, HALF), lambda i: (i % NBLK, 0))] * 2,
        out_shape=[jax.ShapeDtypeStruct((N, HALF), jnp.float32)] * 2,
        scratch_shapes=[pltpu.VMEM((8, D), jnp.float32)],
    )(a0, a1, dA, dB, b, g, be, W)


def _head_body(a0_ref, a1_ref, dA_ref, dB_ref, b_ref, g_ref, be_ref,
               bt_ref, wf1_ref, bf1_ref, wf2_ref, bf2_ref, o_ref,
               st_acc, pool_acc, cnt_acc):
    i = pl.program_id(0)
    dinv = _dinv_of(dA_ref, dB_ref)
    z = _z_of(a0_ref, a1_ref, dinv, b_ref)
    _accum_stats(i, z, st_acc)

    @pl.when(i >= NBLK)
    def _():
        y = _bn_relu(z, g_ref, be_ref, st_acc[...])
        bt = jnp.broadcast_to(bt_ref[...][0], (G, BR))
        oh = (bt == lax.broadcasted_iota(jnp.int32, (G, BR), 0)).astype(
            jnp.float32)
        pool_part = _dot(oh, y)
        cnt_part = jnp.broadcast_to(
            jnp.sum(oh, axis=1, keepdims=True), (G, DEGS))

        @pl.when(i == NBLK)
        def _():
            pool_acc[...] = pool_part
            cnt_acc[...] = cnt_part

        @pl.when(i > NBLK)
        def _():
            pool_acc[...] += pool_part
            cnt_acc[...] += cnt_part

        @pl.when(i == 2 * NBLK - 1)
        def _():
            p = pool_acc[...] / jnp.maximum(cnt_acc[:, :1], 1.0)
            q = jnp.maximum(_dot(p, wf1_ref[...]) + bf1_ref[...], 0.0)
            o_ref[...] = _dot(q, wf2_ref[...]) + bf2_ref[...]


def _head(a0, a1, dA, dB, b, g, be, bt3, Wf1, bf1, Wf2, bf2):
    return pl.pallas_call(
        _head_body,
        grid=(2 * NBLK,),
        in_specs=[
            pl.BlockSpec((BR, HALF), lambda i: (i % NBLK, 0)),
            pl.BlockSpec((BR, HALF), lambda i: (i % NBLK, 0)),
            pl.BlockSpec((BR, DEGS), lambda i: (i % NBLK, 0)),
            pl.BlockSpec((BR, DEGS), lambda i: (i % NBLK, 0)),
            pl.BlockSpec((1, D), lambda i: (0, 0)),
            pl.BlockSpec((1, D), lambda i: (0, 0)),
            pl.BlockSpec((1, D), lambda i: (0, 0)),
            pl.BlockSpec((1, 1, BR), lambda i: (i % NBLK, 0, 0)),
            pl.BlockSpec((D, D), lambda i: (0, 0)),
            pl.BlockSpec((1, D), lambda i: (0, 0)),
            pl.BlockSpec((D, 1), lambda i: (0, 0)),
            pl.BlockSpec((1, 1), lambda i: (0, 0)),
        ],
        out_specs=pl.BlockSpec((G, 1), lambda i: (0, 0)),
        out_shape=jax.ShapeDtypeStruct((G, 1), jnp.float32),
        scratch_shapes=[
            pltpu.VMEM((8, D), jnp.float32),
            pltpu.VMEM((G, D), jnp.float32),
            pltpu.VMEM((G, DEGS), jnp.float32),
        ],
    )(a0, a1, dA, dB, b, g, be, bt3, Wf1, bf1, Wf2, bf2)


def kernel(x, edge_index, batch, W1, b1, g1, be1, W2, b2, g2, be2,
           W3, b3, g3, be3, Wf1, bf1, Wf2, bf2):
    src2d = edge_index[0].reshape(ROWS_E, CHUNK)
    dst2d = edge_index[1].reshape(ROWS_E, CHUNK)
    zeros_nd = jnp.zeros((N, DEGW), jnp.float32)
    ones_cd = jnp.ones((CHUNK, DEGW), jnp.float32)
    bt3 = batch.reshape(NBLK, 1, BR)
    b1r, g1r, be1r = b1.reshape(1, D), g1.reshape(1, D), be1.reshape(1, D)
    b2r, g2r, be2r = b2.reshape(1, D), g2.reshape(1, D), be2.reshape(1, D)
    b3r, g3r, be3r = b3.reshape(1, D), g3.reshape(1, D), be3.reshape(1, D)

    dA, dB = _deg_kernel(dst2d, zeros_nd, ones_cd)
    dA, dB = dA[:, :DEGS], dB[:, :DEGS]

    hs0, hs1 = _mm1(x, W1, dA, dB)
    a0, a1 = _agg_kernel(hs0, hs1, src2d, dst2d)

    hs0, hs1 = _mmn(a0, a1, dA, dB, b1r, g1r, be1r, W2)
    a0, a1 = _agg_kernel(hs0, hs1, src2d, dst2d)

    hs0, hs1 = _mmn(a0, a1, dA, dB, b2r, g2r, be2r, W3)
    a0, a1 = _agg_kernel(hs0, hs1, src2d, dst2d)

    return _head(a0, a1, dA, dB, b3r, g3r, be3r, bt3,
                 Wf1, bf1.reshape(1, D), Wf2, bf2.reshape(1, 1))


# speedup vs baseline: 1.1498x; 1.0168x over previous
import functools

import jax
import jax.numpy as jnp
from jax import lax
from jax.experimental import pallas as pl
from jax.experimental.pallas import tpu as pltpu
from jax.experimental.pallas import tpu_sc as plsc

N = 10000
E = 160000
D = 256
G = 64
EPS = 1e-5

NC = 2
NS = 16
CHUNK = 125
GRP = 40
ROWS_E = E // CHUNK
RPT_AGG = ROWS_E // NS
RPT_DEG = ROWS_E // (NC * NS)
NPT = 624
TAIL0 = NS * NPT
TAILN = N - TAIL0
DEGW = 128
DEGS = 8
HALF = D // 2
BR = 5000
NBLK = N // BR

_sc_mesh = plsc.VectorSubcoreMesh(
    core_axis_name="c", subcore_axis_name="s", num_cores=NC, num_subcores=NS)


def _part_copy(src, dst, s):
    r0 = s * NPT
    pltpu.sync_copy(src.at[pl.ds(r0, NPT)], dst.at[pl.ds(r0, NPT)])

    @pl.when(s == NS - 1)
    def _():
        pltpu.sync_copy(src.at[pl.ds(TAIL0, TAILN)], dst.at[pl.ds(TAIL0, TAILN)])


@functools.partial(
    pl.kernel,
    out_type=(jax.ShapeDtypeStruct((N, DEGW), jnp.float32),
              jax.ShapeDtypeStruct((N, DEGW), jnp.float32)),
    mesh=_sc_mesh,
    scratch_types=[
        pltpu.VMEM((RPT_DEG, CHUNK), jnp.int32),
        pltpu.VMEM((CHUNK, DEGW), jnp.float32),
        pltpu.VMEM_SHARED((N, DEGW), jnp.float32),
    ],
)
def _deg_kernel(dst_hbm, zeros_hbm, ones_hbm, out0, out1, idx_v, ones_v, acc):
    c = lax.axis_index("c")
    s = lax.axis_index("s")
    w = s * NC + c
    _part_copy(zeros_hbm, acc, s)
    pltpu.sync_copy(ones_hbm, ones_v)
    pltpu.sync_copy(dst_hbm.at[pl.ds(w * RPT_DEG, RPT_DEG)], idx_v)
    plsc.subcore_barrier()

    def body(j, carry):
        pltpu.sync_copy(ones_v, acc.at[idx_v.at[j]], add=True)
        return carry

    lax.fori_loop(0, RPT_DEG, body, 0)
    plsc.subcore_barrier()

    @pl.when(c == 0)
    def _():
        _part_copy(acc, out0, s)

    @pl.when(c == 1)
    def _():
        _part_copy(acc, out1, s)


@functools.partial(
    pl.kernel,
    out_type=(jax.ShapeDtypeStruct((N, HALF), jnp.float32),
              jax.ShapeDtypeStruct((N, HALF), jnp.float32)),
    mesh=_sc_mesh,
    scratch_types=[
        pltpu.VMEM((GRP, CHUNK), jnp.int32),
        pltpu.VMEM((GRP, CHUNK), jnp.int32),
        pltpu.VMEM((2, CHUNK, HALF), jnp.float32),
        pltpu.VMEM_SHARED((N, HALF), jnp.float32),
        pltpu.SemaphoreType.DMA,
        pltpu.SemaphoreType.DMA,
    ],
)
def _agg_kernel(hs0, hs1, src_hbm, dst_hbm, out0, out1,
                src_v, dst_v, rows_v, acc, sem0, sem1):
    c = lax.axis_index("c")
    s = lax.axis_index("s")
    e0 = s * RPT_AGG

    def _start(hs_ref):
        pltpu.sync_copy(src_hbm.at[pl.ds(e0, GRP)], src_v)
        pltpu.sync_copy(dst_hbm.at[pl.ds(e0, GRP)], dst_v)
        pltpu.async_copy(hs_ref.at[src_v.at[0]], rows_v.at[0], sem0)

    @pl.when(c == 0)
    def _():
        _start(hs0)
        _part_copy(hs0, acc, s)

    @pl.when(c == 1)
    def _():
        _start(hs1)
        _part_copy(hs1, acc, s)

    plsc.subcore_barrier()

    def _pipe(hs_ref):
        def group(g, carry):
            @pl.when(g > 0)
            def _():
                pltpu.sync_copy(src_hbm.at[pl.ds(e0 + g * GRP, GRP)], src_v)
                pltpu.sync_copy(dst_hbm.at[pl.ds(e0 + g * GRP, GRP)], dst_v)
                pltpu.async_copy(hs_ref.at[src_v.at[0]], rows_v.at[0], sem0)

            def pair(p, carry2):
                j = 2 * p
                pltpu.make_async_copy(
                    hs_ref.at[src_v.at[j]], rows_v.at[0], sem0).wait()
                pltpu.async_copy(hs_ref.at[src_v.at[j + 1]], rows_v.at[1], sem1)
                pltpu.sync_copy(rows_v.at[0], acc.at[dst_v.at[j]], add=True)

                pltpu.make_async_copy(
                    hs_ref.at[src_v.at[j + 1]], rows_v.at[1], sem1).wait()

                @pl.when(p < GRP // 2 - 1)
                def _():
                    pltpu.async_copy(
                        hs_ref.at[src_v.at[j + 2]], rows_v.at[0], sem0)

                pltpu.sync_copy(rows_v.at[1], acc.at[dst_v.at[j + 1]], add=True)
                return carry2

            lax.fori_loop(0, GRP // 2, pair, 0)
            return carry

        lax.fori_loop(0, RPT_AGG // GRP, group, 0)

    @pl.when(c == 0)
    def _():
        _pipe(hs0)

    @pl.when(c == 1)
    def _():
        _pipe(hs1)

    plsc.subcore_barrier()

    @pl.when(c == 0)
    def _():
        _part_copy(acc, out0, s)

    @pl.when(c == 1)
    def _():
        _part_copy(acc, out1, s)


def _dot(a, b):
    return jnp.dot(a, b, preferred_element_type=jnp.float32)


def _dinv_of(dA_ref, dB_ref):
    deg = dA_ref[:, :1] + dB_ref[:, :1] + 1.0
    return lax.rsqrt(deg)


def _mm1_body(x_ref, w_ref, dA_ref, dB_ref, o0_ref, o1_ref):
    dinv = _dinv_of(dA_ref, dB_ref)
    h = _dot(x_ref[...], w_ref[...])
    hs = h * dinv
    o0_ref[...] = hs[:, :HALF]
    o1_ref[...] = hs[:, HALF:]


def _mm1(x, W, dA, dB):
    return pl.pallas_call(
        _mm1_body,
        grid=(NBLK,),
        in_specs=[
            pl.BlockSpec((BR, D), lambda i: (i, 0)),
            pl.BlockSpec((D, D), lambda i: (0, 0)),
            pl.BlockSpec((BR, DEGS), lambda i: (i, 0)),
            pl.BlockSpec((BR, DEGS), lambda i: (i, 0)),
        ],
        out_specs=[pl.BlockSpec((BR, HALF), lambda i: (i, 0))] * 2,
        out_shape=[jax.ShapeDtypeStruct((N, HALF), jnp.float32)] * 2,
    )(x, W, dA, dB)


def _z_of(a0_ref, a1_ref, dinv, b_ref):
    z = jnp.concatenate([a0_ref[...], a1_ref[...]], axis=1)
    return z * dinv + b_ref[...]


def _stats_part(z):
    ps = jnp.sum(z, axis=0, keepdims=True)
    pss = jnp.sum(z * z, axis=0, keepdims=True)
    return jnp.concatenate([ps, pss, jnp.zeros((6, D), jnp.float32)], axis=0)


def _accum_stats(i, z, st_acc):
    @pl.when(i == 0)
    def _():
        st_acc[...] = _stats_part(z)

    @pl.when((i > 0) & (i < NBLK))
    def _():
        st_acc[...] += _stats_part(z)


def _bn_relu(z, g_ref, be_ref, st):
    m = st[0:1, :] * (1.0 / N)
    v = st[1:2, :] * (1.0 / N) - m * m
    return jnp.maximum(g_ref[...] * (z - m) * lax.rsqrt(v + EPS) + be_ref[...], 0.0)


def _mmn_body(a0_ref, a1_ref, dA_ref, dB_ref, b_ref, g_ref, be_ref,
              w_ref, o0_ref, o1_ref, st_acc):
    i = pl.program_id(0)
    dinv = _dinv_of(dA_ref, dB_ref)
    z = _z_of(a0_ref, a1_ref, dinv, b_ref)
    _accum_stats(i, z, st_acc)

    @pl.when(i >= NBLK)
    def _():
        y = _bn_relu(z, g_ref, be_ref, st_acc[...])
        h = _dot(y, w_ref[...])
        hs = h * dinv
        o0_ref[...] = hs[:, :HALF]
        o1_ref[...] = hs[:, HALF:]


def _mmn(a0, a1, dA, dB, b, g, be, W):
    return pl.pallas_call(
        _mmn_body,
        grid=(2 * NBLK,),
        in_specs=[
            pl.BlockSpec((BR, HALF), lambda i: (i % NBLK, 0)),
            pl.BlockSpec((BR, HALF), lambda i: (i % NBLK, 0)),
            pl.BlockSpec((BR, DEGS), lambda i: (i % NBLK, 0)),
            pl.BlockSpec((BR, DEGS), lambda i: (i % NBLK, 0)),
            pl.BlockSpec((1, D), lambda i: (0, 0)),
            pl.BlockSpec((1, D), lambda i: (0, 0)),
            pl.BlockSpec((1, D), lambda i: (0, 0)),
            pl.BlockSpec((D, D), lambda i: (0, 0)),
        ],
        out_specs=[pl.BlockSpec((BR, HALF), lambda i: (i % NBLK, 0))] * 2,
        out_shape=[jax.ShapeDtypeStruct((N, HALF), jnp.float32)] * 2,
        scratch_shapes=[pltpu.VMEM((8, D), jnp.float32)],
    )(a0, a1, dA, dB, b, g, be, W)


def _head_body(a0_ref, a1_ref, dA_ref, dB_ref, b_ref, g_ref, be_ref,
               bt_ref, wf1_ref, bf1_ref, wf2_ref, bf2_ref, o_ref,
               st_acc, pool_acc, cnt_acc):
    i = pl.program_id(0)
    dinv = _dinv_of(dA_ref, dB_ref)
    z = _z_of(a0_ref, a1_ref, dinv, b_ref)
    _accum_stats(i, z, st_acc)

    @pl.when(i >= NBLK)
    def _():
        y = _bn_relu(z, g_ref, be_ref, st_acc[...])
        bt = jnp.broadcast_to(bt_ref[...][0], (G, BR))
        oh = (bt == lax.broadcasted_iota(jnp.int32, (G, BR), 0)).astype(
            jnp.float32)
        pool_part = _dot(oh, y)
        cnt_part = jnp.broadcast_to(
            jnp.sum(oh, axis=1, keepdims=True), (G, DEGS))

        @pl.when(i == NBLK)
        def _():
            pool_acc[...] = pool_part
            cnt_acc[...] = cnt_part

        @pl.when(i > NBLK)
        def _():
            pool_acc[...] += pool_part
            cnt_acc[...] += cnt_part

        @pl.when(i == 2 * NBLK - 1)
        def _():
            p = pool_acc[...] / jnp.maximum(cnt_acc[:, :1], 1.0)
            q = jnp.maximum(_dot(p, wf1_ref[...]) + bf1_ref[...], 0.0)
            o_ref[...] = _dot(q, wf2_ref[...]) + bf2_ref[...]


def _head(a0, a1, dA, dB, b, g, be, bt3, Wf1, bf1, Wf2, bf2):
    return pl.pallas_call(
        _head_body,
        grid=(2 * NBLK,),
        in_specs=[
            pl.BlockSpec((BR, HALF), lambda i: (i % NBLK, 0)),
            pl.BlockSpec((BR, HALF), lambda i: (i % NBLK, 0)),
            pl.BlockSpec((BR, DEGS), lambda i: (i % NBLK, 0)),
            pl.BlockSpec((BR, DEGS), lambda i: (i % NBLK, 0)),
            pl.BlockSpec((1, D), lambda i: (0, 0)),
            pl.BlockSpec((1, D), lambda i: (0, 0)),
            pl.BlockSpec((1, D), lambda i: (0, 0)),
            pl.BlockSpec((1, 1, BR), lambda i: (i % NBLK, 0, 0)),
            pl.BlockSpec((D, D), lambda i: (0, 0)),
            pl.BlockSpec((1, D), lambda i: (0, 0)),
            pl.BlockSpec((D, 1), lambda i: (0, 0)),
            pl.BlockSpec((1, 1), lambda i: (0, 0)),
        ],
        out_specs=pl.BlockSpec((G, 1), lambda i: (0, 0)),
        out_shape=jax.ShapeDtypeStruct((G, 1), jnp.float32),
        scratch_shapes=[
            pltpu.VMEM((8, D), jnp.float32),
            pltpu.VMEM((G, D), jnp.float32),
            pltpu.VMEM((G, DEGS), jnp.float32),
        ],
    )(a0, a1, dA, dB, b, g, be, bt3, Wf1, bf1, Wf2, bf2)


def kernel(x, edge_index, batch, W1, b1, g1, be1, W2, b2, g2, be2,
           W3, b3, g3, be3, Wf1, bf1, Wf2, bf2):
    src2d = edge_index[0].reshape(ROWS_E, CHUNK)
    dst2d = edge_index[1].reshape(ROWS_E, CHUNK)
    zeros_nd = jnp.zeros((N, DEGW), jnp.float32)
    ones_cd = jnp.ones((CHUNK, DEGW), jnp.float32)
    bt3 = batch.reshape(NBLK, 1, BR)
    b1r, g1r, be1r = b1.reshape(1, D), g1.reshape(1, D), be1.reshape(1, D)
    b2r, g2r, be2r = b2.reshape(1, D), g2.reshape(1, D), be2.reshape(1, D)
    b3r, g3r, be3r = b3.reshape(1, D), g3.reshape(1, D), be3.reshape(1, D)

    dA, dB = _deg_kernel(dst2d, zeros_nd, ones_cd)
    dA, dB = dA[:, :DEGS], dB[:, :DEGS]

    hs0, hs1 = _mm1(x, W1, dA, dB)
    a0, a1 = _agg_kernel(hs0, hs1, src2d, dst2d)

    hs0, hs1 = _mmn(a0, a1, dA, dB, b1r, g1r, be1r, W2)
    a0, a1 = _agg_kernel(hs0, hs1, src2d, dst2d)

    hs0, hs1 = _mmn(a0, a1, dA, dB, b2r, g2r, be2r, W3)
    a0, a1 = _agg_kernel(hs0, hs1, src2d, dst2d)

    return _head(a0, a1, dA, dB, b3r, g3r, be3r, bt3,
                 Wf1, bf1.reshape(1, D), Wf2, bf2.reshape(1, 1))
